# Initial kernel scaffold; baseline (speedup 1.0000x reference)
import jax
import jax.numpy as jnp
from jax.experimental import pallas as pl


def kernel(logits):
    raise NotImplementedError("write your pallas kernel here")


# speedup vs baseline: 13.6790x; 13.6790x over previous
import jax
import jax.numpy as jnp
import numpy as np
from jax.experimental import pallas as pl
from jax.experimental.pallas import tpu as pltpu

N = 4096
K = 64
BLOCK_ROWS = 512
INT32_MIN = np.int32(-2147483648)
INT32_MAX = np.int32(2147483647)


def _block_kernel(x_ref, o_ref):
    i = pl.program_id(0)
    x = x_ref[...]
    r = x.shape[0]
    bits = jax.lax.bitcast_convert_type(x, jnp.int32)
    key = bits ^ (jax.lax.shift_right_arithmetic(bits, 31) & jnp.int32(0x7FFFFFFF))
    row = jax.lax.broadcasted_iota(jnp.int32, (r, N), 0) + i * r
    col = jax.lax.broadcasted_iota(jnp.int32, (r, N), 1)
    is_diag = row == col
    key = jnp.where(is_diag, INT32_MAX, key)

    def body(_, carry):
        lo, hi = carry
        mid = lo + jax.lax.shift_right_logical(hi - lo, 1)
        cnt = jnp.sum(jnp.where(key >= mid, jnp.int32(1), jnp.int32(0)),
                      axis=1, keepdims=True)
        ge = cnt >= K
        return jnp.where(ge, mid, lo), jnp.where(ge, hi, mid)

    lo0 = jnp.full((r, 1), INT32_MIN)
    hi0 = jnp.full((r, 1), INT32_MAX)
    thr, _ = jax.lax.fori_loop(0, 32, body, (lo0, hi0))

    a = jax.nn.sigmoid(x)
    a = jnp.where(is_diag, jnp.float32(1.0), a)
    o_ref[...] = jnp.where(key >= thr, a, jnp.float32(0.0))


@jax.jit
def kernel(logits):
    grid = (N // BLOCK_ROWS,)
    return pl.pallas_call(
        _block_kernel,
        grid=grid,
        in_specs=[pl.BlockSpec((BLOCK_ROWS, N), lambda i: (i, 0))],
        out_specs=pl.BlockSpec((BLOCK_ROWS, N), lambda i: (i, 0)),
        out_shape=jax.ShapeDtypeStruct((N, N), jnp.float32),
        compiler_params=pltpu.CompilerParams(
            dimension_semantics=("arbitrary",),
        ),
    )(logits)
